# baseline (speedup 1.0000x reference)
import functools

import jax
import jax.numpy as jnp
from jax import lax
from jax.experimental import pallas as pl
from jax.experimental.pallas import tpu as pltpu
from jax.experimental.pallas import tpu_sc as plsc

MAX_LEN = 8192


def _sc_broadcast_rows(b, s, d):
    info = plsc.get_sparse_core_info()
    nw = info.num_cores * info.num_subcores
    assert s % nw == 0, (s, nw)
    rows_per_w = s // nw
    ch = rows_per_w
    while ch * d * 4 > 128 * 1024:
        ch //= 2
    n_ch = rows_per_w // ch
    nbuf = min(2, n_ch)

    mesh = plsc.VectorSubcoreMesh(core_axis_name="c", subcore_axis_name="s")

    @functools.partial(
        pl.kernel,
        mesh=mesh,
        out_type=jax.ShapeDtypeStruct((b, s, d), jnp.float32),
        scratch_types=[
            pltpu.VMEM((nbuf, ch, d), jnp.float32),
            pltpu.SemaphoreType.DMA((nbuf,)),
            pltpu.SemaphoreType.DMA((nbuf,)),
        ],
    )
    def prog(emb_hbm, out_hbm, buf, lsem, ssem):
        wid = lax.axis_index("s") * info.num_cores + lax.axis_index("c")
        base0 = wid * rows_per_w

        def load(c, slot):
            return pltpu.async_copy(
                emb_hbm.at[pl.ds(base0 + c * ch, ch)], buf.at[slot], lsem.at[slot]
            )

        def stores(c, slot):
            base = base0 + c * ch
            return [
                pltpu.async_copy(
                    buf.at[slot], out_hbm.at[bb, pl.ds(base, ch)], ssem.at[slot]
                )
                for bb in range(b)
            ]

        load_h = {0: load(0, 0)}
        store_h = {}
        for c in range(n_ch):
            slot = c % nbuf
            load_h[c].wait()
            store_h[c] = stores(c, slot)
            nc = c + 1
            if nc < n_ch:
                if nc - nbuf >= 0:
                    for h in store_h[nc - nbuf]:
                        h.wait()
                load_h[nc] = load(nc, nc % nbuf)
        for c in range(max(0, n_ch - nbuf), n_ch):
            for h in store_h[c]:
                h.wait()

    return prog


def kernel(x, emb_weight):
    b, s, _ = x.shape
    d = emb_weight.shape[1]
    if s >= MAX_LEN:
        raise NotImplementedError("s >= MAX_SEQ_LEN not exercised by this problem")
    prog = _sc_broadcast_rows(b, s, d)
    return prog(emb_weight)


# speedup vs baseline: 1.0284x; 1.0284x over previous
import functools

import jax
import jax.numpy as jnp
from jax import lax
from jax.experimental import pallas as pl
from jax.experimental.pallas import tpu as pltpu
from jax.experimental.pallas import tpu_sc as plsc

MAX_LEN = 8192


def _sc_broadcast_rows(b, s, d):
    info = plsc.get_sparse_core_info()
    nw = info.num_cores * info.num_subcores
    assert s % nw == 0, (s, nw)
    rows_per_w = s // nw
    ch = rows_per_w
    while ch * d * 4 > 256 * 1024:
        ch //= 2
    n_ch = rows_per_w // ch
    nbuf = 1

    mesh = plsc.VectorSubcoreMesh(core_axis_name="c", subcore_axis_name="s")

    @functools.partial(
        pl.kernel,
        mesh=mesh,
        out_type=jax.ShapeDtypeStruct((b, s, d), jnp.float32),
        scratch_types=[
            pltpu.VMEM((nbuf, ch, d), jnp.float32),
            pltpu.SemaphoreType.DMA((nbuf,)),
            pltpu.SemaphoreType.DMA((nbuf,)),
        ],
    )
    def prog(emb_hbm, out_hbm, buf, lsem, ssem):
        wid = lax.axis_index("s") * info.num_cores + lax.axis_index("c")
        base0 = wid * rows_per_w

        def load(c, slot):
            return pltpu.async_copy(
                emb_hbm.at[pl.ds(base0 + c * ch, ch)], buf.at[slot], lsem.at[slot]
            )

        def stores(c, slot):
            base = base0 + c * ch
            return [
                pltpu.async_copy(
                    buf.at[slot], out_hbm.at[bb, pl.ds(base, ch)], ssem.at[slot]
                )
                for bb in range(b)
            ]

        load_h = {0: load(0, 0)}
        store_h = {}
        for c in range(n_ch):
            slot = c % nbuf
            load_h[c].wait()
            store_h[c] = stores(c, slot)
            nc = c + 1
            if nc < n_ch:
                if nc - nbuf >= 0:
                    for h in store_h[nc - nbuf]:
                        h.wait()
                load_h[nc] = load(nc, nc % nbuf)
        for c in range(max(0, n_ch - nbuf), n_ch):
            for h in store_h[c]:
                h.wait()

    return prog


def kernel(x, emb_weight):
    b, s, _ = x.shape
    d = emb_weight.shape[1]
    if s >= MAX_LEN:
        raise NotImplementedError("s >= MAX_SEQ_LEN not exercised by this problem")
    prog = _sc_broadcast_rows(b, s, d)
    return prog(emb_weight)
